# Initial kernel scaffold; baseline (speedup 1.0000x reference)
import jax
import jax.numpy as jnp
from jax.experimental import pallas as pl


def kernel(x, edge_index, W1, b1, W2, b2):
    raise NotImplementedError("write your pallas kernel here")


# speedup vs baseline: 7.8666x; 7.8666x over previous
import functools

import jax
import jax.numpy as jnp
from jax import lax
from jax.experimental import pallas as pl
from jax.experimental.pallas import tpu as pltpu
from jax.experimental.pallas import tpu_sc as plsc

N = 10000
NP = 10112
E = 320000
NC, NS = 2, 16
NW = NC * NS
CH = 128
EPS = 10240
EPAD = NW * EPS
NCH = EPS // CH
RPS = NP // NS
DEGW = 16

f32 = jnp.float32


def _sc_mesh():
    return plsc.VectorSubcoreMesh(
        core_axis_name="c", subcore_axis_name="s", num_cores=NC, num_subcores=NS
    )


@functools.partial(
    pl.kernel,
    out_type=jax.ShapeDtypeStruct((NC, NP, DEGW), f32),
    mesh=_sc_mesh(),
    scratch_types=[
        pltpu.VMEM((CH,), jnp.int32),
        pltpu.VMEM((CH, DEGW), f32),
        pltpu.VMEM_SHARED((NP, DEGW), f32),
    ],
)
def _sc_degree(dst_hbm, ones_hbm, zeros_hbm, out_hbm, dst_v, ones_v, acc):
    cid = lax.axis_index("c")
    sid = lax.axis_index("s")
    r0 = sid * RPS
    pltpu.sync_copy(zeros_hbm.at[pl.ds(r0, RPS)], acc.at[pl.ds(r0, RPS)])
    pltpu.sync_copy(ones_hbm, ones_v)
    plsc.subcore_barrier()
    base = (cid * NS + sid) * EPS

    def body(c, carry):
        off = base + c * CH
        pltpu.sync_copy(dst_hbm.at[pl.ds(off, CH)], dst_v)
        pltpu.sync_copy(ones_v, acc.at[dst_v], add=True)
        return carry

    lax.fori_loop(0, NCH, body, 0)
    plsc.subcore_barrier()
    pltpu.sync_copy(acc.at[pl.ds(r0, RPS)], out_hbm.at[cid].at[pl.ds(r0, RPS)])


def _make_sc_scatter(F, tc_tiling=True):
    @functools.partial(
        pl.kernel,
        out_type=jax.ShapeDtypeStruct((NC, NP, F), f32),
        mesh=_sc_mesh(),
        compiler_params=pltpu.CompilerParams(use_tc_tiling_on_sc=tc_tiling),
        scratch_types=[
            pltpu.VMEM((CH,), jnp.int32),
            pltpu.VMEM((CH,), jnp.int32),
            pltpu.VMEM((CH, F), f32),
            pltpu.VMEM_SHARED((NP, F), f32),
            pltpu.SemaphoreType.DMA,
        ],
    )
    def scat(mh_hbm, src_hbm, dst_hbm, zeros_hbm, out_hbm,
             src_v, dst_v, rows_v, acc, sem):
        cid = lax.axis_index("c")
        sid = lax.axis_index("s")
        r0 = sid * RPS
        pltpu.sync_copy(zeros_hbm.at[pl.ds(r0, RPS)], acc.at[pl.ds(r0, RPS)])
        plsc.subcore_barrier()
        base = (cid * NS + sid) * EPS

        def body(c, carry):
            off = base + c * CH
            pltpu.sync_copy(src_hbm.at[pl.ds(off, CH)], src_v)
            pltpu.async_copy(mh_hbm.at[src_v], rows_v, sem).wait()
            pltpu.sync_copy(dst_hbm.at[pl.ds(off, CH)], dst_v)
            pltpu.sync_copy(rows_v, acc.at[dst_v], add=True)
            return carry

        lax.fori_loop(0, NCH, body, 0)
        plsc.subcore_barrier()
        pltpu.sync_copy(acc.at[pl.ds(r0, RPS)], out_hbm.at[cid].at[pl.ds(r0, RPS)])

    return scat


_sc_scatter128 = _make_sc_scatter(128)
_sc_scatter64 = _make_sc_scatter(64, tc_tiling=False)


RB = 1000
GRID = N // RB


def _tc_layer1(degpair, x, W1):
    def body(dp_ref, x_ref, w_ref, mh_ref, dinv_ref):
        deg = dp_ref[0, :, :1] + dp_ref[1, :, :1] + 1.0
        dinv = lax.rsqrt(deg)
        dinv_ref[...] = dinv
        m = jnp.dot(x_ref[...], w_ref[...], preferred_element_type=f32)
        mh_ref[...] = dinv * m

    return pl.pallas_call(
        body,
        grid=(GRID,),
        in_specs=[
            pl.BlockSpec((2, RB, DEGW), lambda i: (0, i, 0)),
            pl.BlockSpec((RB, 128), lambda i: (i, 0)),
            pl.BlockSpec((128, 128), lambda i: (0, 0)),
        ],
        out_specs=[
            pl.BlockSpec((RB, 128), lambda i: (i, 0)),
            pl.BlockSpec((RB, 1), lambda i: (i, 0)),
        ],
        out_shape=[
            jax.ShapeDtypeStruct((N, 128), f32),
            jax.ShapeDtypeStruct((N, 1), f32),
        ],
    )(degpair, x, W1)


def _tc_layer2(s1, mh1, dinv, b1, W2):
    def body(s_ref, mh_ref, dinv_ref, b_ref, w_ref, out_ref):
        s = s_ref[0] + s_ref[1] + mh_ref[...]
        h = jnp.maximum(dinv_ref[...] * s + b_ref[...], 0.0)
        out_ref[...] = dinv_ref[...] * jnp.dot(
            h, w_ref[...], preferred_element_type=f32)

    return pl.pallas_call(
        body,
        grid=(GRID,),
        in_specs=[
            pl.BlockSpec((2, RB, 128), lambda i: (0, i, 0)),
            pl.BlockSpec((RB, 128), lambda i: (i, 0)),
            pl.BlockSpec((RB, 1), lambda i: (i, 0)),
            pl.BlockSpec((1, 128), lambda i: (0, 0)),
            pl.BlockSpec((128, 64), lambda i: (0, 0)),
        ],
        out_specs=pl.BlockSpec((RB, 64), lambda i: (i, 0)),
        out_shape=jax.ShapeDtypeStruct((N, 64), f32),
    )(s1, mh1, dinv, b1, W2)


def _tc_final(s2, mh2, dinv, b2):
    def body(s_ref, mh_ref, dinv_ref, b_ref, out_ref):
        s = s_ref[0] + s_ref[1] + mh_ref[...]
        out_ref[...] = dinv_ref[...] * s + b_ref[...]

    return pl.pallas_call(
        body,
        grid=(GRID,),
        in_specs=[
            pl.BlockSpec((2, RB, 64), lambda i: (0, i, 0)),
            pl.BlockSpec((RB, 64), lambda i: (i, 0)),
            pl.BlockSpec((RB, 1), lambda i: (i, 0)),
            pl.BlockSpec((1, 64), lambda i: (0, 0)),
        ],
        out_specs=pl.BlockSpec((RB, 64), lambda i: (i, 0)),
        out_shape=jax.ShapeDtypeStruct((N, 64), f32),
    )(s2, mh2, dinv, b2)


def kernel(x, edge_index, W1, b1, W2, b2):
    src = edge_index[0].astype(jnp.int32)
    dst = edge_index[1].astype(jnp.int32)
    pad = EPAD - E
    src_p = jnp.concatenate([src, jnp.zeros((pad,), jnp.int32)])
    dst_p = jnp.concatenate([dst, jnp.full((pad,), N, jnp.int32)])

    ones_ch = jnp.ones((CH, DEGW), f32)
    zeros_deg = jnp.zeros((NP, DEGW), f32)
    zeros_f128 = jnp.zeros((NP, 128), f32)
    zeros_f64 = jnp.zeros((NP, 64), f32)

    degpair = _sc_degree(dst_p, ones_ch, zeros_deg)
    mh1, dinv = _tc_layer1(degpair, x, W1)
    s1 = _sc_scatter128(mh1, src_p, dst_p, zeros_f128)
    mh2 = _tc_layer2(s1, mh1, dinv, b1.reshape(1, 128), W2)
    s2 = _sc_scatter64(mh2, src_p, dst_p, zeros_f64)
    z = _tc_final(s2, mh2, dinv, b2.reshape(1, 64))
    return z


# speedup vs baseline: 12.2204x; 1.5535x over previous
import functools

import jax
import jax.numpy as jnp
from jax import lax
from jax.experimental import pallas as pl
from jax.experimental.pallas import tpu as pltpu
from jax.experimental.pallas import tpu_sc as plsc

N = 10000
NP = 10112
E = 320000
NC, NS = 2, 16
NW = NC * NS
CH = 128
EPS = 10240
EPAD = NW * EPS
NCH = EPS // CH
RPS = NP // NS
DEGW = 16

f32 = jnp.float32


def _sc_mesh():
    return plsc.VectorSubcoreMesh(
        core_axis_name="c", subcore_axis_name="s", num_cores=NC, num_subcores=NS
    )


NBUF = 4
NGRP = NCH // NBUF


@functools.partial(
    pl.kernel,
    out_type=jax.ShapeDtypeStruct((NC, NP, DEGW), f32),
    mesh=_sc_mesh(),
    scratch_types=[
        pltpu.VMEM((NCH, CH), jnp.int32),
        pltpu.VMEM((CH, DEGW), f32),
        pltpu.VMEM_SHARED((NP, DEGW), f32),
        pltpu.SemaphoreType.DMA,
    ],
)
def _sc_degree(dst_hbm, ones_hbm, zeros_hbm, out_hbm, dst_v, ones_v, acc, ssem):
    cid = lax.axis_index("c")
    sid = lax.axis_index("s")
    wid = cid * NS + sid
    r0 = sid * RPS
    pltpu.sync_copy(zeros_hbm.at[pl.ds(r0, RPS)], acc.at[pl.ds(r0, RPS)])
    pltpu.sync_copy(ones_hbm, ones_v)
    pltpu.sync_copy(dst_hbm.at[pl.ds(wid * NCH, NCH)], dst_v)
    plsc.subcore_barrier()

    def fire(g, carry):
        for b in range(NBUF):
            pltpu.async_copy(ones_v, acc.at[dst_v.at[g * NBUF + b]], ssem,
                             add=True)
        return carry

    lax.fori_loop(0, NGRP, fire, 0)

    def drain(g, carry):
        for _ in range(NBUF):
            pltpu.make_async_copy(ones_v, acc.at[dst_v.at[0]], ssem).wait()
        return carry

    lax.fori_loop(0, NGRP, drain, 0)
    plsc.subcore_barrier()
    pltpu.sync_copy(acc.at[pl.ds(r0, RPS)], out_hbm.at[cid].at[pl.ds(r0, RPS)])


ECH = EPAD // NS
SCH = ECH // CH
SGRP = SCH // NBUF


def _make_sc_scatter(H):
    @functools.partial(
        pl.kernel,
        out_type=jax.ShapeDtypeStruct((NC, NP, H), f32),
        mesh=_sc_mesh(),
        compiler_params=pltpu.CompilerParams(use_tc_tiling_on_sc=False),
        scratch_types=[
            pltpu.VMEM((ECH,), jnp.int32),
            pltpu.VMEM((SCH, CH), jnp.int32),
            pltpu.VMEM((NBUF, CH, H), f32),
            pltpu.VMEM_SHARED((NP, H), f32),
            pltpu.SemaphoreType.DMA,
            pltpu.SemaphoreType.DMA,
            pltpu.SemaphoreType.DMA,
            pltpu.SemaphoreType.DMA,
            pltpu.SemaphoreType.DMA,
            pltpu.SemaphoreType.DMA,
            pltpu.SemaphoreType.DMA,
            pltpu.SemaphoreType.DMA,
        ],
    )
    def scat(mh_hbm, src_hbm, dst_hbm, zeros_hbm, out_hbm,
             src_v, dst_v, rows_v, acc,
             gs0, gs1, gs2, gs3, ss0, ss1, ss2, ss3):
        gsem = [gs0, gs1, gs2, gs3]
        ssem = [ss0, ss1, ss2, ss3]
        cid = lax.axis_index("c")
        sid = lax.axis_index("s")
        r0 = sid * RPS
        pltpu.sync_copy(zeros_hbm.at[pl.ds(r0, RPS)], acc.at[pl.ds(r0, RPS)])
        pltpu.sync_copy(src_hbm.at[pl.ds(sid * ECH, ECH)], src_v)
        pltpu.sync_copy(dst_hbm.at[pl.ds(sid * SCH, SCH)], dst_v)

        def xform(i, carry):
            v = src_v[pl.ds(i * 16, 16)]
            src_v[pl.ds(i * 16, 16)] = v + v + cid
            return carry

        lax.fori_loop(0, ECH // 16, xform, 0)
        plsc.subcore_barrier()

        def fire_gather(c, b):
            pltpu.async_copy(mh_hbm.at[src_v.at[pl.ds(c * CH, CH)]],
                             rows_v.at[b], gsem[b])

        def wait_gather(b, c=0):
            pltpu.make_async_copy(mh_hbm.at[src_v.at[pl.ds(c * CH, CH)]],
                                  rows_v.at[b], gsem[b]).wait()

        def fire_scatter(c, b):
            pltpu.async_copy(rows_v.at[b], acc.at[dst_v.at[c]], ssem[b],
                             add=True)

        def wait_scatter(b):
            pltpu.make_async_copy(rows_v.at[b], acc.at[dst_v.at[0]],
                                  ssem[b]).wait()

        for j in range(NBUF):
            fire_gather(j, j)
            wait_gather(j)
            fire_scatter(j, j)

        def group(g, carry):
            for j in range(NBUF):
                c = g * NBUF + j
                b = j
                wait_scatter(b)
                fire_gather(c, b)
                wait_gather(b)
                fire_scatter(c, b)
            return carry

        lax.fori_loop(1, SGRP, group, 0)
        for j in range(NBUF):
            wait_scatter(j)
        plsc.subcore_barrier()
        pltpu.sync_copy(acc.at[pl.ds(r0, RPS)], out_hbm.at[cid].at[pl.ds(r0, RPS)])

    return scat


_sc_scatter64h = _make_sc_scatter(64)
_sc_scatter32h = _make_sc_scatter(32)


RB = 1000
GRID = N // RB


def _tc_layer1(degpair, x, W1):
    def body(dp_ref, x_ref, w_ref, mh_ref, dinv_ref):
        deg = dp_ref[0, :, :1] + dp_ref[1, :, :1] + 1.0
        dinv = lax.rsqrt(deg)
        dinv_ref[...] = dinv
        m = jnp.dot(x_ref[...], w_ref[...], preferred_element_type=f32)
        mh_ref[...] = dinv * m

    return pl.pallas_call(
        body,
        grid=(GRID,),
        in_specs=[
            pl.BlockSpec((2, RB, DEGW), lambda i: (0, i, 0)),
            pl.BlockSpec((RB, 128), lambda i: (i, 0)),
            pl.BlockSpec((128, 128), lambda i: (0, 0)),
        ],
        out_specs=[
            pl.BlockSpec((RB, 128), lambda i: (i, 0)),
            pl.BlockSpec((RB, 1), lambda i: (i, 0)),
        ],
        out_shape=[
            jax.ShapeDtypeStruct((N, 128), f32),
            jax.ShapeDtypeStruct((N, 1), f32),
        ],
    )(degpair, x, W1)


def _tc_layer2(s1, mh1, dinv, b1, W2):
    def body(s_ref, mh_ref, dinv_ref, b_ref, w_ref, out_ref):
        s = jnp.concatenate([s_ref[0], s_ref[1]], axis=1) + mh_ref[...]
        h = jnp.maximum(dinv_ref[...] * s + b_ref[...], 0.0)
        out_ref[...] = dinv_ref[...] * jnp.dot(
            h, w_ref[...], preferred_element_type=f32)

    return pl.pallas_call(
        body,
        grid=(GRID,),
        in_specs=[
            pl.BlockSpec((2, RB, 64), lambda i: (0, i, 0)),
            pl.BlockSpec((RB, 128), lambda i: (i, 0)),
            pl.BlockSpec((RB, 1), lambda i: (i, 0)),
            pl.BlockSpec((1, 128), lambda i: (0, 0)),
            pl.BlockSpec((128, 64), lambda i: (0, 0)),
        ],
        out_specs=pl.BlockSpec((RB, 64), lambda i: (i, 0)),
        out_shape=jax.ShapeDtypeStruct((N, 64), f32),
    )(s1, mh1, dinv, b1, W2)


def _tc_final(s2, mh2, dinv, b2):
    def body(s_ref, mh_ref, dinv_ref, b_ref, out_ref):
        s = jnp.concatenate([s_ref[0], s_ref[1]], axis=1) + mh_ref[...]
        out_ref[...] = dinv_ref[...] * s + b_ref[...]

    return pl.pallas_call(
        body,
        grid=(GRID,),
        in_specs=[
            pl.BlockSpec((2, RB, 32), lambda i: (0, i, 0)),
            pl.BlockSpec((RB, 64), lambda i: (i, 0)),
            pl.BlockSpec((RB, 1), lambda i: (i, 0)),
            pl.BlockSpec((1, 64), lambda i: (0, 0)),
        ],
        out_specs=pl.BlockSpec((RB, 64), lambda i: (i, 0)),
        out_shape=jax.ShapeDtypeStruct((N, 64), f32),
    )(s2, mh2, dinv, b2)


def kernel(x, edge_index, W1, b1, W2, b2):
    src = edge_index[0].astype(jnp.int32)
    dst = edge_index[1].astype(jnp.int32)
    pad = EPAD - E
    src_p = jnp.concatenate([src, jnp.zeros((pad,), jnp.int32)])
    dst_p = jnp.concatenate([dst, jnp.full((pad,), N, jnp.int32)])
    dst_2d = dst_p.reshape(NS * SCH, CH)

    ones_ch = jnp.ones((CH, DEGW), f32)
    zeros_deg = jnp.zeros((NP, DEGW), f32)
    zeros_h64 = jnp.zeros((NP, 64), f32)
    zeros_h32 = jnp.zeros((NP, 32), f32)

    degpair = _sc_degree(dst_2d, ones_ch, zeros_deg)
    mh1, dinv = _tc_layer1(degpair, x, W1)
    s1 = _sc_scatter64h(mh1.reshape(2 * N, 64), src_p, dst_2d, zeros_h64)
    mh2 = _tc_layer2(s1, mh1, dinv, b1.reshape(1, 128), W2)
    s2 = _sc_scatter32h(mh2.reshape(2 * N, 32), src_p, dst_2d, zeros_h32)
    z = _tc_final(s2, mh2, dinv, b2.reshape(1, 64))
    return z
